# Initial kernel scaffold; baseline (speedup 1.0000x reference)
import jax
import jax.numpy as jnp
from jax.experimental import pallas as pl


def kernel(x_p, x_d, edge_attr_p, edge_attr_d, edge_index_p, edge_index_d, x_p_batch, x_d_batch, Wp0, bp0, Wp1, bp1, Wp2, bp2, Wd0, bd0, Wd1, bd1, Wd2, bd2, lin0_w, lin0_b, lin1_w, lin1_b):
    raise NotImplementedError("write your pallas kernel here")


# speedup vs baseline: 5.5970x; 5.5970x over previous
import jax
import jax.numpy as jnp
from jax import lax
from jax.experimental import pallas as pl
from jax.experimental.pallas import tpu as pltpu
from jax.experimental.pallas import tpu_sc as plsc

N = 10000
E = 320000
EMB = 256
G = 64

NC = 2
NS = 16
CHUNK = 128
CB = 32
CPT = 160
PER_TILE = CPT * CHUNK
E_PAD = NS * PER_TILE
SINK = N
ACC_ROWS = N + 8
STRIPE = 624
TAIL = N - NS * STRIPE
DEG_ROWS = 16 * 640

R = 1000
NBLK = N // R

_sc_mesh = plsc.VectorSubcoreMesh(core_axis_name="c", subcore_axis_name="s")


def _deg_body(dstp, dstd, degp, degd, idx_v, ones_v, acc):
    c = lax.axis_index("c")
    s = lax.axis_index("s")

    def _fill(i, _):
        for k in range(8):
            ones_v[i, pl.ds(16 * k, 16)] = jnp.ones((16,), jnp.float32)
        return 0
    lax.fori_loop(0, CHUNK, _fill, 0, unroll=False)

    for k in range(5):
        pltpu.sync_copy(ones_v, acc.at[pl.ds(s * 640 + 128 * k, 128)])
    plsc.subcore_barrier()

    def _scatter(dref):
        def _bb(bk, _):
            pltpu.sync_copy(dref.at[s].at[pl.ds(bk * CB, CB)], idx_v)

            def _eb(j, _):
                pltpu.sync_copy(ones_v, acc.at[idx_v.at[j]], add=True)
                return 0
            lax.fori_loop(0, CB, _eb, 0, unroll=False)
            return 0
        lax.fori_loop(0, CPT // CB, _bb, 0, unroll=False)

    @pl.when(c == 0)
    def _():
        _scatter(dstp)

    @pl.when(c == 1)
    def _():
        _scatter(dstd)

    plsc.subcore_barrier()

    def _copy_out(oref):
        pltpu.sync_copy(acc.at[pl.ds(s * STRIPE, STRIPE)],
                        oref.at[pl.ds(s * STRIPE, STRIPE)])

        @pl.when(s == 0)
        def _():
            pltpu.sync_copy(acc.at[pl.ds(NS * STRIPE, TAIL)],
                            oref.at[pl.ds(NS * STRIPE, TAIL)])

    @pl.when(c == 0)
    def _():
        _copy_out(degp)

    @pl.when(c == 1)
    def _():
        _copy_out(degd)


_deg_call = pl.kernel(
    _deg_body,
    out_type=(jax.ShapeDtypeStruct((N, 128), jnp.float32),
              jax.ShapeDtypeStruct((N, 128), jnp.float32)),
    mesh=_sc_mesh,
    scratch_types=[
        pltpu.VMEM((CB, CHUNK), jnp.int32),
        pltpu.VMEM((CHUNK, 128), jnp.float32),
        pltpu.VMEM_SHARED((DEG_ROWS, 128), jnp.float32),
    ],
    name="gcn_degree_sc",
)


def _spmm_body(u2, srci, dsti, out2, idx_s, idx_d, rows0, rows1, acc,
               sem0, sem1):
    c = lax.axis_index("c")
    s = lax.axis_index("s")
    uh = u2.at[c]
    pltpu.sync_copy(uh.at[pl.ds(s * STRIPE, STRIPE)],
                    acc.at[pl.ds(s * STRIPE, STRIPE)])

    @pl.when(s == 0)
    def _():
        pltpu.sync_copy(uh.at[pl.ds(NS * STRIPE, TAIL)],
                        acc.at[pl.ds(NS * STRIPE, TAIL)])
        pltpu.sync_copy(uh.at[pl.ds(0, 8)], acc.at[pl.ds(N, 8)])

    plsc.subcore_barrier()

    def _bb(bk, _):
        pltpu.sync_copy(srci.at[s].at[pl.ds(bk * CB, CB)], idx_s)
        pltpu.sync_copy(dsti.at[s].at[pl.ds(bk * CB, CB)], idx_d)

        def _body(j, _):
            a = 2 * j
            b = 2 * j + 1
            cpa = pltpu.async_copy(uh.at[idx_s.at[a]], rows0, sem0)
            cpb = pltpu.async_copy(uh.at[idx_s.at[b]], rows1, sem1)
            cpa.wait()
            pltpu.sync_copy(rows0, acc.at[idx_d.at[a]], add=True)
            cpb.wait()
            pltpu.sync_copy(rows1, acc.at[idx_d.at[b]], add=True)
            return 0
        lax.fori_loop(0, CB // 2, _body, 0, unroll=False)
        return 0
    lax.fori_loop(0, CPT // CB, _bb, 0, unroll=False)

    plsc.subcore_barrier()
    pltpu.sync_copy(acc.at[pl.ds(s * STRIPE, STRIPE)],
                    out2.at[c].at[pl.ds(s * STRIPE, STRIPE)])

    @pl.when(s == 0)
    def _():
        pltpu.sync_copy(acc.at[pl.ds(NS * STRIPE, TAIL)],
                        out2.at[c].at[pl.ds(NS * STRIPE, TAIL)])


_spmm_call = pl.kernel(
    _spmm_body,
    out_type=jax.ShapeDtypeStruct((2, N, 128), jnp.float32),
    mesh=_sc_mesh,
    scratch_types=[
        pltpu.VMEM((CB, CHUNK), jnp.int32),
        pltpu.VMEM((CB, CHUNK), jnp.int32),
        pltpu.VMEM((CHUNK, 128), jnp.float32),
        pltpu.VMEM((CHUNK, 128), jnp.float32),
        pltpu.VMEM_SHARED((ACC_ROWS, 128), jnp.float32),
        pltpu.SemaphoreType.DMA,
        pltpu.SemaphoreType.DMA,
    ],
    name="gcn_spmm_sc",
)


def _pre_body(x_ref, w_ref, deg_ref, out_ref):
    dinv = lax.rsqrt(deg_ref[:, 0:1])
    u = jnp.dot(x_ref[:], w_ref[:], preferred_element_type=jnp.float32) * dinv
    out_ref[0] = u[:, :128]
    out_ref[1] = u[:, 128:]


def _mid_body(acc_ref, deg_ref, b_ref, w_ref, out_ref):
    dinv = lax.rsqrt(deg_ref[:, 0:1])
    h0 = jnp.maximum(acc_ref[0] * dinv + b_ref[:, :128], 0.0)
    h1 = jnp.maximum(acc_ref[1] * dinv + b_ref[:, 128:], 0.0)
    t = (jnp.dot(h0, w_ref[:128], preferred_element_type=jnp.float32)
         + jnp.dot(h1, w_ref[128:], preferred_element_type=jnp.float32))
    u = t * dinv
    out_ref[0] = u[:, :128]
    out_ref[1] = u[:, 128:]


def _seg_update(acc_ref, deg_ref, b_ref, batch_ref, m_ref):
    dinv = lax.rsqrt(deg_ref[:, 0:1])
    h = jnp.concatenate([acc_ref[0], acc_ref[1]], axis=1) * dinv + b_ref[:]
    h = jnp.maximum(h, 0.0)
    ids = batch_ref[:]
    g0 = jnp.min(ids)
    g1 = jnp.max(ids)

    def _gb(g, _):
        msk = (ids == g)
        v = jnp.max(jnp.where(msk, h, -jnp.inf), axis=0, keepdims=True)
        m_ref[pl.ds(g, 1), :] = jnp.maximum(m_ref[pl.ds(g, 1), :], v)
        return 0
    lax.fori_loop(g0, g1 + 1, _gb, 0, unroll=False)


def _post_body(accp_ref, degp_ref, bp_ref, batchp_ref,
               accd_ref, degd_ref, bd_ref, batchd_ref,
               l0w_ref, l0b_ref, l1w_ref, l1b_ref, out_ref, mp_ref, md_ref):
    i = pl.program_id(0)

    @pl.when(i == 0)
    def _():
        mp_ref[:] = jnp.full((G, EMB), -jnp.inf, jnp.float32)
        md_ref[:] = jnp.full((G, EMB), -jnp.inf, jnp.float32)

    _seg_update(accp_ref, degp_ref, bp_ref, batchp_ref, mp_ref)
    _seg_update(accd_ref, degd_ref, bd_ref, batchd_ref, md_ref)

    @pl.when(i == NBLK - 1)
    def _():
        xcat = jnp.concatenate([mp_ref[:], md_ref[:]], axis=1)
        y = (jnp.dot(xcat, l0w_ref[:], preferred_element_type=jnp.float32)
             + l0b_ref[:])
        z = (jnp.dot(y, l1w_ref[:], preferred_element_type=jnp.float32)
             + l1b_ref[:])
        out_ref[:] = z


def _pre_call(x, w, deg):
    return pl.pallas_call(
        _pre_body,
        grid=(NBLK,),
        in_specs=[
            pl.BlockSpec((R, 128), lambda i: (i, 0)),
            pl.BlockSpec((128, EMB), lambda i: (0, 0)),
            pl.BlockSpec((R, 128), lambda i: (i, 0)),
        ],
        out_specs=pl.BlockSpec((2, R, 128), lambda i: (0, i, 0)),
        out_shape=jax.ShapeDtypeStruct((2, N, 128), jnp.float32),
        name="gcn_pre_tc",
    )(x, w, deg)


def _mid_call(acc2, deg, b, w):
    return pl.pallas_call(
        _mid_body,
        grid=(NBLK,),
        in_specs=[
            pl.BlockSpec((2, R, 128), lambda i: (0, i, 0)),
            pl.BlockSpec((R, 128), lambda i: (i, 0)),
            pl.BlockSpec((1, EMB), lambda i: (0, 0)),
            pl.BlockSpec((EMB, EMB), lambda i: (0, 0)),
        ],
        out_specs=pl.BlockSpec((2, R, 128), lambda i: (0, i, 0)),
        out_shape=jax.ShapeDtypeStruct((2, N, 128), jnp.float32),
        name="gcn_mid_tc",
    )(acc2, deg, b, w)


def _post_call(accp, degp, bp, batchp, accd, degd, bd, batchd,
               l0w, l0b, l1w, l1b):
    return pl.pallas_call(
        _post_body,
        grid=(NBLK,),
        in_specs=[
            pl.BlockSpec((2, R, 128), lambda i: (0, i, 0)),
            pl.BlockSpec((R, 128), lambda i: (i, 0)),
            pl.BlockSpec((1, EMB), lambda i: (0, 0)),
            pl.BlockSpec((R, 1), lambda i: (i, 0)),
            pl.BlockSpec((2, R, 128), lambda i: (0, i, 0)),
            pl.BlockSpec((R, 128), lambda i: (i, 0)),
            pl.BlockSpec((1, EMB), lambda i: (0, 0)),
            pl.BlockSpec((R, 1), lambda i: (i, 0)),
            pl.BlockSpec((2 * EMB, 2 * EMB), lambda i: (0, 0)),
            pl.BlockSpec((1, 2 * EMB), lambda i: (0, 0)),
            pl.BlockSpec((2 * EMB, 1), lambda i: (0, 0)),
            pl.BlockSpec((1, 1), lambda i: (0, 0)),
        ],
        out_specs=pl.BlockSpec((G, 1), lambda i: (0, 0)),
        out_shape=jax.ShapeDtypeStruct((G, 1), jnp.float32),
        scratch_shapes=[
            pltpu.VMEM((G, EMB), jnp.float32),
            pltpu.VMEM((G, EMB), jnp.float32),
        ],
        name="gcn_post_tc",
    )(accp, degp, bp, batchp, accd, degd, bd, batchd, l0w, l0b, l1w, l1b)


def _pad_edges(ei):
    src = jnp.concatenate(
        [ei[0], jnp.zeros((E_PAD - E,), jnp.int32)]).reshape(NS, CPT, CHUNK)
    dst = jnp.concatenate(
        [ei[1], jnp.full((E_PAD - E,), SINK, jnp.int32)]).reshape(NS, CPT, CHUNK)
    return src, dst


def kernel(x_p, x_d, edge_attr_p, edge_attr_d, edge_index_p, edge_index_d,
           x_p_batch, x_d_batch, Wp0, bp0, Wp1, bp1, Wp2, bp2,
           Wd0, bd0, Wd1, bd1, Wd2, bd2, lin0_w, lin0_b, lin1_w, lin1_b):
    srcp, dstp = _pad_edges(edge_index_p)
    srcd, dstd = _pad_edges(edge_index_d)
    degp, degd = _deg_call(dstp, dstd)

    up = _pre_call(x_p, Wp0, degp)
    ud = _pre_call(x_d, Wd0, degd)
    for bp_l, wp_n, bd_l, wd_n in ((bp0, Wp1, bd0, Wd1), (bp1, Wp2, bd1, Wd2)):
        accp = _spmm_call(up, srcp, dstp)
        accd = _spmm_call(ud, srcd, dstd)
        up = _mid_call(accp, degp, bp_l.reshape(1, EMB), wp_n)
        ud = _mid_call(accd, degd, bd_l.reshape(1, EMB), wd_n)
    accp = _spmm_call(up, srcp, dstp)
    accd = _spmm_call(ud, srcd, dstd)

    return _post_call(
        accp, degp, bp2.reshape(1, EMB), x_p_batch.reshape(N, 1),
        accd, degd, bd2.reshape(1, EMB), x_d_batch.reshape(N, 1),
        lin0_w, lin0_b.reshape(1, 2 * EMB), lin1_w, lin1_b.reshape(1, 1))


# speedup vs baseline: 5.9958x; 1.0713x over previous
import jax
import jax.numpy as jnp
from jax import lax
from jax.experimental import pallas as pl
from jax.experimental.pallas import tpu as pltpu
from jax.experimental.pallas import tpu_sc as plsc

N = 10000
E = 320000
EMB = 256
G = 64

NC = 2
NS = 16
CHUNK = 128
CB = 40
CPT = 160
PER_TILE = CPT * CHUNK
E_PAD = NS * PER_TILE
SINK = N
ACC_ROWS = N + 8
STRIPE = 624
TAIL = N - NS * STRIPE
DEG_ROWS = 16 * 640

R = 1000
NBLK = N // R

_sc_mesh = plsc.VectorSubcoreMesh(core_axis_name="c", subcore_axis_name="s")


def _deg_body(dstp, dstd, degp, degd, idx_v, ones_v, acc, dsem):
    c = lax.axis_index("c")
    s = lax.axis_index("s")

    def _fill(i, _):
        for k in range(8):
            ones_v[i, pl.ds(16 * k, 16)] = jnp.ones((16,), jnp.float32)
        return 0
    lax.fori_loop(0, CHUNK, _fill, 0, unroll=False)

    for k in range(5):
        pltpu.sync_copy(ones_v, acc.at[pl.ds(s * 640 + 128 * k, 128)])
    plsc.subcore_barrier()

    def _scatter(dref):
        def _bb(bk, _):
            pltpu.sync_copy(dref.at[s].at[pl.ds(bk * CB, CB)], idx_v)

            def _fire(j, _):
                pltpu.async_copy(ones_v, acc.at[idx_v.at[j]], dsem, add=True)
                return 0
            lax.fori_loop(0, CB, _fire, 0, unroll=False)

            def _drain(j, _):
                pltpu.make_async_copy(ones_v, acc.at[idx_v.at[j]], dsem).wait()
                return 0
            lax.fori_loop(0, CB, _drain, 0, unroll=False)
            return 0
        lax.fori_loop(0, CPT // CB, _bb, 0, unroll=False)

    @pl.when(c == 0)
    def _():
        _scatter(dstp)

    @pl.when(c == 1)
    def _():
        _scatter(dstd)

    plsc.subcore_barrier()

    def _copy_out(oref):
        pltpu.sync_copy(acc.at[pl.ds(s * STRIPE, STRIPE)],
                        oref.at[pl.ds(s * STRIPE, STRIPE)])

        @pl.when(s == 0)
        def _():
            pltpu.sync_copy(acc.at[pl.ds(NS * STRIPE, TAIL)],
                            oref.at[pl.ds(NS * STRIPE, TAIL)])

    @pl.when(c == 0)
    def _():
        _copy_out(degp)

    @pl.when(c == 1)
    def _():
        _copy_out(degd)


_deg_call = pl.kernel(
    _deg_body,
    out_type=(jax.ShapeDtypeStruct((N, 128), jnp.float32),
              jax.ShapeDtypeStruct((N, 128), jnp.float32)),
    mesh=_sc_mesh,
    scratch_types=[
        pltpu.VMEM((CB, CHUNK), jnp.int32),
        pltpu.VMEM((CHUNK, 128), jnp.float32),
        pltpu.VMEM_SHARED((DEG_ROWS, 128), jnp.float32),
        pltpu.SemaphoreType.DMA,
    ],
    name="gcn_degree_sc",
)


def _spmm_body(u2, srci, dsti, out2, idx_s, idx_d, rows0, rows1, acc,
               gs0, gs1, ss0, ss1):
    c = lax.axis_index("c")
    s = lax.axis_index("s")
    uh = u2.at[c]
    pltpu.sync_copy(uh.at[pl.ds(s * STRIPE, STRIPE)],
                    acc.at[pl.ds(s * STRIPE, STRIPE)])

    @pl.when(s == 0)
    def _():
        pltpu.sync_copy(uh.at[pl.ds(NS * STRIPE, TAIL)],
                        acc.at[pl.ds(NS * STRIPE, TAIL)])
        pltpu.sync_copy(uh.at[pl.ds(0, 8)], acc.at[pl.ds(N, 8)])

    plsc.subcore_barrier()

    def _bb(bk, _):
        pltpu.sync_copy(srci.at[s].at[pl.ds(bk * CB, CB)], idx_s)
        pltpu.sync_copy(dsti.at[s].at[pl.ds(bk * CB, CB)], idx_d)
        pltpu.async_copy(uh.at[idx_s.at[0]], rows0, gs0)

        def _pair(j, _):
            k0 = 2 * j
            k1 = 2 * j + 1
            pltpu.make_async_copy(uh.at[idx_s.at[k0]], rows0, gs0).wait()
            pltpu.async_copy(rows0, acc.at[idx_d.at[k0]], ss0, add=True)

            @pl.when(j > 0)
            def _():
                pltpu.make_async_copy(rows1, acc.at[idx_d.at[k0]], ss1).wait()
            pltpu.async_copy(uh.at[idx_s.at[k1]], rows1, gs1)

            pltpu.make_async_copy(uh.at[idx_s.at[k1]], rows1, gs1).wait()
            pltpu.async_copy(rows1, acc.at[idx_d.at[k1]], ss1, add=True)
            pltpu.make_async_copy(rows0, acc.at[idx_d.at[k0]], ss0).wait()

            @pl.when(j < CB // 2 - 1)
            def _():
                pltpu.async_copy(uh.at[idx_s.at[k1 + 1]], rows0, gs0)
            return 0
        lax.fori_loop(0, CB // 2, _pair, 0, unroll=False)
        pltpu.make_async_copy(rows1, acc.at[idx_d.at[CB - 1]], ss1).wait()
        return 0
    lax.fori_loop(0, CPT // CB, _bb, 0, unroll=False)

    plsc.subcore_barrier()
    pltpu.sync_copy(acc.at[pl.ds(s * STRIPE, STRIPE)],
                    out2.at[c].at[pl.ds(s * STRIPE, STRIPE)])

    @pl.when(s == 0)
    def _():
        pltpu.sync_copy(acc.at[pl.ds(NS * STRIPE, TAIL)],
                        out2.at[c].at[pl.ds(NS * STRIPE, TAIL)])


_spmm_call = pl.kernel(
    _spmm_body,
    out_type=jax.ShapeDtypeStruct((2, N, 128), jnp.float32),
    mesh=_sc_mesh,
    scratch_types=[
        pltpu.VMEM((CB, CHUNK), jnp.int32),
        pltpu.VMEM((CB, CHUNK), jnp.int32),
        pltpu.VMEM((CHUNK, 128), jnp.float32),
        pltpu.VMEM((CHUNK, 128), jnp.float32),
        pltpu.VMEM_SHARED((ACC_ROWS, 128), jnp.float32),
        pltpu.SemaphoreType.DMA,
        pltpu.SemaphoreType.DMA,
        pltpu.SemaphoreType.DMA,
        pltpu.SemaphoreType.DMA,
    ],
    name="gcn_spmm_sc",
)


def _pre_body(x_ref, w_ref, deg_ref, out_ref):
    dinv = lax.rsqrt(deg_ref[:, 0:1])
    u = jnp.dot(x_ref[:], w_ref[:], preferred_element_type=jnp.float32) * dinv
    out_ref[0] = u[:, :128]
    out_ref[1] = u[:, 128:]


def _mid_body(acc_ref, deg_ref, b_ref, w_ref, out_ref):
    dinv = lax.rsqrt(deg_ref[:, 0:1])
    h0 = jnp.maximum(acc_ref[0] * dinv + b_ref[:, :128], 0.0)
    h1 = jnp.maximum(acc_ref[1] * dinv + b_ref[:, 128:], 0.0)
    t = (jnp.dot(h0, w_ref[:128], preferred_element_type=jnp.float32)
         + jnp.dot(h1, w_ref[128:], preferred_element_type=jnp.float32))
    u = t * dinv
    out_ref[0] = u[:, :128]
    out_ref[1] = u[:, 128:]


def _seg_update(acc_ref, deg_ref, b_ref, batch_ref, m_ref):
    dinv = lax.rsqrt(deg_ref[:, 0:1])
    h = jnp.concatenate([acc_ref[0], acc_ref[1]], axis=1) * dinv + b_ref[:]
    h = jnp.maximum(h, 0.0)
    ids = batch_ref[:]
    g0 = jnp.min(ids)
    g1 = jnp.max(ids)

    def _gb(g, _):
        msk = (ids == g)
        v = jnp.max(jnp.where(msk, h, -jnp.inf), axis=0, keepdims=True)
        m_ref[pl.ds(g, 1), :] = jnp.maximum(m_ref[pl.ds(g, 1), :], v)
        return 0
    lax.fori_loop(g0, g1 + 1, _gb, 0, unroll=False)


def _post_body(accp_ref, degp_ref, bp_ref, batchp_ref,
               accd_ref, degd_ref, bd_ref, batchd_ref,
               l0w_ref, l0b_ref, l1w_ref, l1b_ref, out_ref, mp_ref, md_ref):
    i = pl.program_id(0)

    @pl.when(i == 0)
    def _():
        mp_ref[:] = jnp.full((G, EMB), -jnp.inf, jnp.float32)
        md_ref[:] = jnp.full((G, EMB), -jnp.inf, jnp.float32)

    _seg_update(accp_ref, degp_ref, bp_ref, batchp_ref, mp_ref)
    _seg_update(accd_ref, degd_ref, bd_ref, batchd_ref, md_ref)

    @pl.when(i == NBLK - 1)
    def _():
        xcat = jnp.concatenate([mp_ref[:], md_ref[:]], axis=1)
        y = (jnp.dot(xcat, l0w_ref[:], preferred_element_type=jnp.float32)
             + l0b_ref[:])
        z = (jnp.dot(y, l1w_ref[:], preferred_element_type=jnp.float32)
             + l1b_ref[:])
        out_ref[:] = z


def _pre_call(x, w, deg):
    return pl.pallas_call(
        _pre_body,
        grid=(NBLK,),
        in_specs=[
            pl.BlockSpec((R, 128), lambda i: (i, 0)),
            pl.BlockSpec((128, EMB), lambda i: (0, 0)),
            pl.BlockSpec((R, 128), lambda i: (i, 0)),
        ],
        out_specs=pl.BlockSpec((2, R, 128), lambda i: (0, i, 0)),
        out_shape=jax.ShapeDtypeStruct((2, N, 128), jnp.float32),
        name="gcn_pre_tc",
    )(x, w, deg)


def _mid_call(acc2, deg, b, w):
    return pl.pallas_call(
        _mid_body,
        grid=(NBLK,),
        in_specs=[
            pl.BlockSpec((2, R, 128), lambda i: (0, i, 0)),
            pl.BlockSpec((R, 128), lambda i: (i, 0)),
            pl.BlockSpec((1, EMB), lambda i: (0, 0)),
            pl.BlockSpec((EMB, EMB), lambda i: (0, 0)),
        ],
        out_specs=pl.BlockSpec((2, R, 128), lambda i: (0, i, 0)),
        out_shape=jax.ShapeDtypeStruct((2, N, 128), jnp.float32),
        name="gcn_mid_tc",
    )(acc2, deg, b, w)


def _post_call(accp, degp, bp, batchp, accd, degd, bd, batchd,
               l0w, l0b, l1w, l1b):
    return pl.pallas_call(
        _post_body,
        grid=(NBLK,),
        in_specs=[
            pl.BlockSpec((2, R, 128), lambda i: (0, i, 0)),
            pl.BlockSpec((R, 128), lambda i: (i, 0)),
            pl.BlockSpec((1, EMB), lambda i: (0, 0)),
            pl.BlockSpec((R, 1), lambda i: (i, 0)),
            pl.BlockSpec((2, R, 128), lambda i: (0, i, 0)),
            pl.BlockSpec((R, 128), lambda i: (i, 0)),
            pl.BlockSpec((1, EMB), lambda i: (0, 0)),
            pl.BlockSpec((R, 1), lambda i: (i, 0)),
            pl.BlockSpec((2 * EMB, 2 * EMB), lambda i: (0, 0)),
            pl.BlockSpec((1, 2 * EMB), lambda i: (0, 0)),
            pl.BlockSpec((2 * EMB, 1), lambda i: (0, 0)),
            pl.BlockSpec((1, 1), lambda i: (0, 0)),
        ],
        out_specs=pl.BlockSpec((G, 1), lambda i: (0, 0)),
        out_shape=jax.ShapeDtypeStruct((G, 1), jnp.float32),
        scratch_shapes=[
            pltpu.VMEM((G, EMB), jnp.float32),
            pltpu.VMEM((G, EMB), jnp.float32),
        ],
        name="gcn_post_tc",
    )(accp, degp, bp, batchp, accd, degd, bd, batchd, l0w, l0b, l1w, l1b)


def _pad_edges(ei):
    src = jnp.concatenate(
        [ei[0], jnp.zeros((E_PAD - E,), jnp.int32)]).reshape(NS, CPT, CHUNK)
    dst = jnp.concatenate(
        [ei[1], jnp.full((E_PAD - E,), SINK, jnp.int32)]).reshape(NS, CPT, CHUNK)
    return src, dst


def kernel(x_p, x_d, edge_attr_p, edge_attr_d, edge_index_p, edge_index_d,
           x_p_batch, x_d_batch, Wp0, bp0, Wp1, bp1, Wp2, bp2,
           Wd0, bd0, Wd1, bd1, Wd2, bd2, lin0_w, lin0_b, lin1_w, lin1_b):
    srcp, dstp = _pad_edges(edge_index_p)
    srcd, dstd = _pad_edges(edge_index_d)
    degp, degd = _deg_call(dstp, dstd)

    up = _pre_call(x_p, Wp0, degp)
    ud = _pre_call(x_d, Wd0, degd)
    for bp_l, wp_n, bd_l, wd_n in ((bp0, Wp1, bd0, Wd1), (bp1, Wp2, bd1, Wd2)):
        accp = _spmm_call(up, srcp, dstp)
        accd = _spmm_call(ud, srcd, dstd)
        up = _mid_call(accp, degp, bp_l.reshape(1, EMB), wp_n)
        ud = _mid_call(accd, degd, bd_l.reshape(1, EMB), wd_n)
    accp = _spmm_call(up, srcp, dstp)
    accd = _spmm_call(ud, srcd, dstd)

    return _post_call(
        accp, degp, bp2.reshape(1, EMB), x_p_batch.reshape(N, 1),
        accd, degd, bd2.reshape(1, EMB), x_d_batch.reshape(N, 1),
        lin0_w, lin0_b.reshape(1, 2 * EMB), lin1_w, lin1_b.reshape(1, 1))


# speedup vs baseline: 6.2432x; 1.0413x over previous
import jax
import jax.numpy as jnp
from jax import lax
from jax.experimental import pallas as pl
from jax.experimental.pallas import tpu as pltpu
from jax.experimental.pallas import tpu_sc as plsc

N = 10000
E = 320000
EMB = 256
G = 64

NC = 2
NS = 16
CHUNK = 128
CB = 40
CPT = 160
PER_TILE = CPT * CHUNK
E_PAD = NS * PER_TILE
SINK = N
ACC_ROWS = N + 8
STRIPE = 624
TAIL = N - NS * STRIPE
DEG_ROWS = 16 * 640

R = 1000
NBLK = N // R

_sc_mesh = plsc.VectorSubcoreMesh(core_axis_name="c", subcore_axis_name="s")


def _deg_body(dstp, dstd, degp, degd, idx_v, ones_v, acc, dsem):
    c = lax.axis_index("c")
    s = lax.axis_index("s")

    def _fill(i, _):
        for k in range(8):
            ones_v[i, pl.ds(16 * k, 16)] = jnp.ones((16,), jnp.float32)
        return 0
    lax.fori_loop(0, CHUNK, _fill, 0, unroll=False)

    for k in range(5):
        pltpu.sync_copy(ones_v, acc.at[pl.ds(s * 640 + 128 * k, 128)])
    plsc.subcore_barrier()

    def _scatter(dref):
        def _bb(bk, _):
            pltpu.sync_copy(dref.at[s].at[pl.ds(bk * CB, CB)], idx_v)

            def _fire(j, _):
                pltpu.async_copy(ones_v, acc.at[idx_v.at[j]], dsem, add=True)
                return 0
            lax.fori_loop(0, CB, _fire, 0, unroll=False)

            def _drain(j, _):
                pltpu.make_async_copy(ones_v, acc.at[idx_v.at[j]], dsem).wait()
                return 0
            lax.fori_loop(0, CB, _drain, 0, unroll=False)
            return 0
        lax.fori_loop(0, CPT // CB, _bb, 0, unroll=False)

    @pl.when(c == 0)
    def _():
        _scatter(dstp)

    @pl.when(c == 1)
    def _():
        _scatter(dstd)

    plsc.subcore_barrier()

    def _copy_out(oref):
        pltpu.sync_copy(acc.at[pl.ds(s * STRIPE, STRIPE)],
                        oref.at[pl.ds(s * STRIPE, STRIPE)])

        @pl.when(s == 0)
        def _():
            pltpu.sync_copy(acc.at[pl.ds(NS * STRIPE, TAIL)],
                            oref.at[pl.ds(NS * STRIPE, TAIL)])

    @pl.when(c == 0)
    def _():
        _copy_out(degp)

    @pl.when(c == 1)
    def _():
        _copy_out(degd)


_deg_call = pl.kernel(
    _deg_body,
    out_type=(jax.ShapeDtypeStruct((N, 128), jnp.float32),
              jax.ShapeDtypeStruct((N, 128), jnp.float32)),
    mesh=_sc_mesh,
    scratch_types=[
        pltpu.VMEM((CB, CHUNK), jnp.int32),
        pltpu.VMEM((CHUNK, 128), jnp.float32),
        pltpu.VMEM_SHARED((DEG_ROWS, 128), jnp.float32),
        pltpu.SemaphoreType.DMA,
    ],
    name="gcn_degree_sc",
)


def _spmm_body(u2, srci, dsti, out2, idx_s, idx_d, rows0, rows1, acc,
               gs0, gs1, ss0, ss1):
    c = lax.axis_index("c")
    s = lax.axis_index("s")
    uh = u2.at[c]
    pltpu.sync_copy(uh.at[pl.ds(s * STRIPE, STRIPE)],
                    acc.at[pl.ds(s * STRIPE, STRIPE)])

    @pl.when(s == 0)
    def _():
        pltpu.sync_copy(uh.at[pl.ds(NS * STRIPE, TAIL)],
                        acc.at[pl.ds(NS * STRIPE, TAIL)])
        pltpu.sync_copy(uh.at[pl.ds(0, 8)], acc.at[pl.ds(N, 8)])

    plsc.subcore_barrier()

    def _bb(bk, _):
        pltpu.sync_copy(srci.at[s].at[pl.ds(bk * CB, CB)], idx_s)
        pltpu.sync_copy(dsti.at[s].at[pl.ds(bk * CB, CB)], idx_d)
        pltpu.async_copy(uh.at[idx_s.at[0]], rows0, gs0)

        def _pair(j, _):
            k0 = 2 * j
            k1 = 2 * j + 1
            pltpu.make_async_copy(uh.at[idx_s.at[k0]], rows0, gs0).wait()
            pltpu.async_copy(rows0, acc.at[idx_d.at[k0]], ss0, add=True)

            @pl.when(j > 0)
            def _():
                pltpu.make_async_copy(rows1, acc.at[idx_d.at[k0]], ss1).wait()
            pltpu.async_copy(uh.at[idx_s.at[k1]], rows1, gs1)

            pltpu.make_async_copy(uh.at[idx_s.at[k1]], rows1, gs1).wait()
            pltpu.async_copy(rows1, acc.at[idx_d.at[k1]], ss1, add=True)
            pltpu.make_async_copy(rows0, acc.at[idx_d.at[k0]], ss0).wait()

            @pl.when(j < CB // 2 - 1)
            def _():
                pltpu.async_copy(uh.at[idx_s.at[k1 + 1]], rows0, gs0)
            return 0
        lax.fori_loop(0, CB // 2, _pair, 0, unroll=False)
        pltpu.make_async_copy(rows1, acc.at[idx_d.at[CB - 1]], ss1).wait()
        return 0
    lax.fori_loop(0, CPT // CB, _bb, 0, unroll=False)

    plsc.subcore_barrier()
    pltpu.sync_copy(acc.at[pl.ds(s * STRIPE, STRIPE)],
                    out2.at[c].at[pl.ds(s * STRIPE, STRIPE)])

    @pl.when(s == 0)
    def _():
        pltpu.sync_copy(acc.at[pl.ds(NS * STRIPE, TAIL)],
                        out2.at[c].at[pl.ds(NS * STRIPE, TAIL)])


_spmm_call = pl.kernel(
    _spmm_body,
    out_type=jax.ShapeDtypeStruct((2, N, 128), jnp.float32),
    mesh=_sc_mesh,
    scratch_types=[
        pltpu.VMEM((CB, CHUNK), jnp.int32),
        pltpu.VMEM((CB, CHUNK), jnp.int32),
        pltpu.VMEM((CHUNK, 128), jnp.float32),
        pltpu.VMEM((CHUNK, 128), jnp.float32),
        pltpu.VMEM_SHARED((ACC_ROWS, 128), jnp.float32),
        pltpu.SemaphoreType.DMA,
        pltpu.SemaphoreType.DMA,
        pltpu.SemaphoreType.DMA,
        pltpu.SemaphoreType.DMA,
    ],
    name="gcn_spmm_sc",
)


def _mm_body(x_ref, w_ref, out_ref):
    u = jnp.dot(x_ref[:], w_ref[:], preferred_element_type=jnp.float32)
    out_ref[0] = u[:, :128]
    out_ref[1] = u[:, 128:]


def _scale_body(t_ref, deg_ref, out_ref):
    dinv = lax.rsqrt(deg_ref[:, 0:1])
    out_ref[0] = t_ref[0] * dinv
    out_ref[1] = t_ref[1] * dinv


def _mid_body(acc_ref, deg_ref, b_ref, w_ref, out_ref):
    dinv = lax.rsqrt(deg_ref[:, 0:1])
    h0 = jnp.maximum(acc_ref[0] * dinv + b_ref[:, :128], 0.0)
    h1 = jnp.maximum(acc_ref[1] * dinv + b_ref[:, 128:], 0.0)
    t = (jnp.dot(h0, w_ref[:128], preferred_element_type=jnp.float32)
         + jnp.dot(h1, w_ref[128:], preferred_element_type=jnp.float32))
    u = t * dinv
    out_ref[0] = u[:, :128]
    out_ref[1] = u[:, 128:]


def _seg_update(acc_ref, deg_ref, b_ref, batch_ref, m_ref):
    dinv = lax.rsqrt(deg_ref[:, 0:1])
    h = jnp.concatenate([acc_ref[0], acc_ref[1]], axis=1) * dinv + b_ref[:]
    h = jnp.maximum(h, 0.0)
    ids = batch_ref[:]
    g0 = jnp.min(ids)
    g1 = jnp.max(ids)

    def _gb(g, _):
        msk = (ids == g)
        v = jnp.max(jnp.where(msk, h, -jnp.inf), axis=0, keepdims=True)
        m_ref[pl.ds(g, 1), :] = jnp.maximum(m_ref[pl.ds(g, 1), :], v)
        return 0
    lax.fori_loop(g0, g1 + 1, _gb, 0, unroll=False)


def _post_body(accp_ref, degp_ref, bp_ref, batchp_ref,
               accd_ref, degd_ref, bd_ref, batchd_ref,
               l0w_ref, l0b_ref, l1w_ref, l1b_ref, out_ref, mp_ref, md_ref):
    i = pl.program_id(0)

    @pl.when(i == 0)
    def _():
        mp_ref[:] = jnp.full((G, EMB), -jnp.inf, jnp.float32)
        md_ref[:] = jnp.full((G, EMB), -jnp.inf, jnp.float32)

    _seg_update(accp_ref, degp_ref, bp_ref, batchp_ref, mp_ref)
    _seg_update(accd_ref, degd_ref, bd_ref, batchd_ref, md_ref)

    @pl.when(i == NBLK - 1)
    def _():
        xcat = jnp.concatenate([mp_ref[:], md_ref[:]], axis=1)
        y = (jnp.dot(xcat, l0w_ref[:], preferred_element_type=jnp.float32)
             + l0b_ref[:])
        z = (jnp.dot(y, l1w_ref[:], preferred_element_type=jnp.float32)
             + l1b_ref[:])
        out_ref[:] = z


def _mm_call(x, w):
    return pl.pallas_call(
        _mm_body,
        grid=(NBLK,),
        in_specs=[
            pl.BlockSpec((R, 128), lambda i: (i, 0)),
            pl.BlockSpec((128, EMB), lambda i: (0, 0)),
        ],
        out_specs=pl.BlockSpec((2, R, 128), lambda i: (0, i, 0)),
        out_shape=jax.ShapeDtypeStruct((2, N, 128), jnp.float32),
        name="gcn_mm_tc",
    )(x, w)


def _scale_call(t, deg):
    return pl.pallas_call(
        _scale_body,
        grid=(NBLK,),
        in_specs=[
            pl.BlockSpec((2, R, 128), lambda i: (0, i, 0)),
            pl.BlockSpec((R, 128), lambda i: (i, 0)),
        ],
        out_specs=pl.BlockSpec((2, R, 128), lambda i: (0, i, 0)),
        out_shape=jax.ShapeDtypeStruct((2, N, 128), jnp.float32),
        name="gcn_scale_tc",
    )(t, deg)


def _mid_call(acc2, deg, b, w):
    return pl.pallas_call(
        _mid_body,
        grid=(NBLK,),
        in_specs=[
            pl.BlockSpec((2, R, 128), lambda i: (0, i, 0)),
            pl.BlockSpec((R, 128), lambda i: (i, 0)),
            pl.BlockSpec((1, EMB), lambda i: (0, 0)),
            pl.BlockSpec((EMB, EMB), lambda i: (0, 0)),
        ],
        out_specs=pl.BlockSpec((2, R, 128), lambda i: (0, i, 0)),
        out_shape=jax.ShapeDtypeStruct((2, N, 128), jnp.float32),
        name="gcn_mid_tc",
    )(acc2, deg, b, w)


def _post_call(accp, degp, bp, batchp, accd, degd, bd, batchd,
               l0w, l0b, l1w, l1b):
    return pl.pallas_call(
        _post_body,
        grid=(NBLK,),
        in_specs=[
            pl.BlockSpec((2, R, 128), lambda i: (0, i, 0)),
            pl.BlockSpec((R, 128), lambda i: (i, 0)),
            pl.BlockSpec((1, EMB), lambda i: (0, 0)),
            pl.BlockSpec((R, 1), lambda i: (i, 0)),
            pl.BlockSpec((2, R, 128), lambda i: (0, i, 0)),
            pl.BlockSpec((R, 128), lambda i: (i, 0)),
            pl.BlockSpec((1, EMB), lambda i: (0, 0)),
            pl.BlockSpec((R, 1), lambda i: (i, 0)),
            pl.BlockSpec((2 * EMB, 2 * EMB), lambda i: (0, 0)),
            pl.BlockSpec((1, 2 * EMB), lambda i: (0, 0)),
            pl.BlockSpec((2 * EMB, 1), lambda i: (0, 0)),
            pl.BlockSpec((1, 1), lambda i: (0, 0)),
        ],
        out_specs=pl.BlockSpec((G, 1), lambda i: (0, 0)),
        out_shape=jax.ShapeDtypeStruct((G, 1), jnp.float32),
        scratch_shapes=[
            pltpu.VMEM((G, EMB), jnp.float32),
            pltpu.VMEM((G, EMB), jnp.float32),
        ],
        name="gcn_post_tc",
    )(accp, degp, bp, batchp, accd, degd, bd, batchd, l0w, l0b, l1w, l1b)


def _pad_edges(ei):
    src = jnp.concatenate(
        [ei[0], jnp.zeros((E_PAD - E,), jnp.int32)]).reshape(NS, CPT, CHUNK)
    dst = jnp.concatenate(
        [ei[1], jnp.full((E_PAD - E,), SINK, jnp.int32)]).reshape(NS, CPT, CHUNK)
    return src, dst


def kernel(x_p, x_d, edge_attr_p, edge_attr_d, edge_index_p, edge_index_d,
           x_p_batch, x_d_batch, Wp0, bp0, Wp1, bp1, Wp2, bp2,
           Wd0, bd0, Wd1, bd1, Wd2, bd2, lin0_w, lin0_b, lin1_w, lin1_b):
    srcp, dstp = _pad_edges(edge_index_p)
    srcd, dstd = _pad_edges(edge_index_d)
    tp = _mm_call(x_p, Wp0)
    td = _mm_call(x_d, Wd0)
    degp, degd = _deg_call(dstp, dstd)

    up = _scale_call(tp, degp)
    ud = _scale_call(td, degd)
    for bp_l, wp_n, bd_l, wd_n in ((bp0, Wp1, bd0, Wd1), (bp1, Wp2, bd1, Wd2)):
        accp = _spmm_call(up, srcp, dstp)
        accd = _spmm_call(ud, srcd, dstd)
        up = _mid_call(accp, degp, bp_l.reshape(1, EMB), wp_n)
        ud = _mid_call(accd, degd, bd_l.reshape(1, EMB), wd_n)
    accp = _spmm_call(up, srcp, dstp)
    accd = _spmm_call(ud, srcd, dstd)

    return _post_call(
        accp, degp, bp2.reshape(1, EMB), x_p_batch.reshape(N, 1),
        accd, degd, bd2.reshape(1, EMB), x_d_batch.reshape(N, 1),
        lin0_w, lin0_b.reshape(1, 2 * EMB), lin1_w, lin1_b.reshape(1, 1))
